# baseline (speedup 1.0000x reference)
import functools

import jax
import jax.numpy as jnp
from jax import lax
from jax.experimental import pallas as pl
from jax.experimental.pallas import tpu as pltpu
from jax.experimental.pallas import tpu_sc as plsc

EMSIZE = 1024
NHEADS = 16
NHID = 2048
NEXP = 8
B = 2
S = 2048
HD = EMSIZE // NHEADS
T = B * S
LN_EPS = 1e-5

PAIRS = 2 * T
BLK_M = 128
N_SLOTS = PAIRS + NEXP * BLK_M
NBLK = N_SLOTS // BLK_M
CH = 512

NCORE = 2
NSUB = 16
NW = NCORE * NSUB
DROW = 32
PPT = PAIRS // NW
DCH = PPT // DROW
TPT = T // NW
GCH = TPT // DROW


def _dot_t(x, w):
    return jax.lax.dot_general(
        x, w, (((1,), (1,)), ((), ())), preferred_element_type=jnp.float32)


def _qkv_body(x_ref, w_ref, b_ref, o_ref):
    o_ref[...] = _dot_t(x_ref[...], w_ref[...]) + b_ref[...]


def _qkv_proj(xt, Wqkv, bqkv):
    blk = 512
    return pl.pallas_call(
        _qkv_body,
        grid=(T // blk,),
        in_specs=[
            pl.BlockSpec((blk, EMSIZE), lambda i: (i, 0)),
            pl.BlockSpec((3 * EMSIZE, EMSIZE), lambda i: (0, 0)),
            pl.BlockSpec((1, 3 * EMSIZE), lambda i: (0, 0)),
        ],
        out_specs=pl.BlockSpec((blk, 3 * EMSIZE), lambda i: (i, 0)),
        out_shape=jax.ShapeDtypeStruct((T, 3 * EMSIZE), jnp.float32),
    )(xt, Wqkv, bqkv.reshape(1, -1))


HPB = 4


def _att_body(q_ref, k_ref, v_ref, o_ref):
    outs = []
    for j in range(HPB):
        q = q_ref[:, j * HD:(j + 1) * HD]
        k = k_ref[:, j * HD:(j + 1) * HD]
        v = v_ref[:, j * HD:(j + 1) * HD]
        att = _dot_t(q, k) * (1.0 / (HD ** 0.5))
        e = jnp.exp(att)
        s = jnp.sum(e, axis=-1, keepdims=True)
        outs.append(jnp.dot(e, v, preferred_element_type=jnp.float32) / s)
    o_ref[...] = jnp.concatenate(outs, axis=-1)


def _attention(qkv):
    blk = 512
    nblk = S // blk
    nhb = NHEADS // HPB
    return pl.pallas_call(
        _att_body,
        grid=(B, nhb, nblk),
        in_specs=[
            pl.BlockSpec((blk, HPB * HD), lambda b, h, i: (b * nblk + i, h)),
            pl.BlockSpec((S, HPB * HD), lambda b, h, i: (b, nhb + h)),
            pl.BlockSpec((S, HPB * HD), lambda b, h, i: (b, 2 * nhb + h)),
        ],
        out_specs=pl.BlockSpec((blk, HPB * HD),
                               lambda b, h, i: (b * nblk + i, h)),
        out_shape=jax.ShapeDtypeStruct((T, EMSIZE), jnp.float32),
    )(qkv, qkv, qkv)


QBLK = 512


def _att_fused_body(x_ref, wq_ref, wk_ref, wv_ref, bq_ref, bk_ref, bv_ref,
                    o_ref, k_scr, v_scr):
    i = pl.program_id(2)

    @pl.when(i == 0)
    def _kv():
        xb = x_ref[...]
        k_scr[...] = _dot_t(xb, wk_ref[...]) + bk_ref[...]
        v_scr[...] = _dot_t(xb, wv_ref[...]) + bv_ref[...]

    q = (_dot_t(x_ref[pl.ds(i * QBLK, QBLK)], wq_ref[...]) + bq_ref[...])
    outs = []
    for j in range(HPB):
        qj = q[:, j * HD:(j + 1) * HD]
        kj = k_scr[:, j * HD:(j + 1) * HD]
        vj = v_scr[:, j * HD:(j + 1) * HD]
        att = _dot_t(qj, kj) * (1.0 / (HD ** 0.5))
        e = jnp.exp(att)
        s = jnp.sum(e, axis=-1, keepdims=True)
        outs.append(jnp.dot(e, vj, preferred_element_type=jnp.float32) / s)
    o_ref[...] = jnp.concatenate(outs, axis=-1)


def _att_fused(xt, Wqkv, bqkv):
    nblk = S // QBLK
    nhb = NHEADS // HPB
    gw = HPB * HD
    b2d = bqkv.reshape(1, 3 * EMSIZE)
    return pl.pallas_call(
        _att_fused_body,
        grid=(B, nhb, nblk),
        in_specs=[
            pl.BlockSpec((S, EMSIZE), lambda b, h, i: (b, 0)),
            pl.BlockSpec((gw, EMSIZE), lambda b, h, i: (h, 0)),
            pl.BlockSpec((gw, EMSIZE), lambda b, h, i: (nhb + h, 0)),
            pl.BlockSpec((gw, EMSIZE), lambda b, h, i: (2 * nhb + h, 0)),
            pl.BlockSpec((1, gw), lambda b, h, i: (0, h)),
            pl.BlockSpec((1, gw), lambda b, h, i: (0, nhb + h)),
            pl.BlockSpec((1, gw), lambda b, h, i: (0, 2 * nhb + h)),
        ],
        out_specs=pl.BlockSpec((QBLK, gw), lambda b, h, i: (b * nblk + i, h)),
        out_shape=jax.ShapeDtypeStruct((T, EMSIZE), jnp.float32),
        scratch_shapes=[
            pltpu.VMEM((S, gw), jnp.float32),
            pltpu.VMEM((S, gw), jnp.float32),
        ],
    )(xt, Wqkv, Wqkv, Wqkv, b2d, b2d, b2d)


def _proj_ln_body(o_ref, w_ref, b_ref, x_ref, g_ref, bb_ref, wg_ref,
                  h_ref, lg_ref):
    y = _dot_t(o_ref[...], w_ref[...]) + b_ref[...] + x_ref[...]
    m = jnp.mean(y, axis=-1, keepdims=True)
    c = y - m
    v = jnp.mean(c * c, axis=-1, keepdims=True)
    h = c * jax.lax.rsqrt(v + LN_EPS) * g_ref[...] + bb_ref[...]
    h_ref[...] = h
    lg_ref[...] = jnp.dot(h, wg_ref[...], preferred_element_type=jnp.float32)


def _proj_ln(o, Wout, bout, xt, g, b, Wg):
    blk = 512
    return pl.pallas_call(
        _proj_ln_body,
        grid=(T // blk,),
        in_specs=[
            pl.BlockSpec((blk, EMSIZE), lambda i: (i, 0)),
            pl.BlockSpec((EMSIZE, EMSIZE), lambda i: (0, 0)),
            pl.BlockSpec((1, EMSIZE), lambda i: (0, 0)),
            pl.BlockSpec((blk, EMSIZE), lambda i: (i, 0)),
            pl.BlockSpec((1, EMSIZE), lambda i: (0, 0)),
            pl.BlockSpec((1, EMSIZE), lambda i: (0, 0)),
            pl.BlockSpec((EMSIZE, NEXP), lambda i: (0, 0)),
        ],
        out_specs=[
            pl.BlockSpec((blk, EMSIZE), lambda i: (i, 0)),
            pl.BlockSpec((blk, NEXP), lambda i: (i, 0)),
        ],
        out_shape=[
            jax.ShapeDtypeStruct((T, EMSIZE), jnp.float32),
            jax.ShapeDtypeStruct((T, NEXP), jnp.float32),
        ],
    )(o, Wout, bout.reshape(1, -1), xt, g.reshape(1, -1), b.reshape(1, -1),
      Wg)


def _router_body(lg_ref, dst_ref, gate_ref, be_ref):
    lg = lg_ref[...]
    mx = jnp.max(lg, axis=-1, keepdims=True)
    ex = jnp.exp(lg - mx)
    probs = ex / jnp.sum(ex, axis=-1, keepdims=True)
    cols = jax.lax.broadcasted_iota(jnp.int32, probs.shape, 1)
    v1 = jnp.max(probs, axis=-1, keepdims=True)
    i1 = jnp.min(jnp.where(probs == v1, cols, NEXP), axis=-1, keepdims=True)
    masked = jnp.where(cols == i1, -1.0, probs)
    v2 = jnp.max(masked, axis=-1, keepdims=True)
    i2 = jnp.min(jnp.where(masked == v2, cols, NEXP), axis=-1, keepdims=True)
    den = v1 + v2 + 1e-9
    g1 = v1 / den
    g2 = v2 / den

    oh1 = (cols == i1).astype(jnp.float32)
    oh2 = (cols == i2).astype(jnp.float32)
    oh = jnp.concatenate([oh1, oh2], axis=0)

    r = jax.lax.broadcasted_iota(jnp.int32, (CH, CH), 0)
    c = jax.lax.broadcasted_iota(jnp.int32, (CH, CH), 1)
    trils = (c < r).astype(jnp.float32)
    run = jnp.zeros((1, NEXP), jnp.float32)
    ranks = []
    for ci in range(PAIRS // CH):
        blk = oh[ci * CH:(ci + 1) * CH]
        ranks.append(
            jnp.dot(trils, blk, preferred_element_type=jnp.float32) + run)
        run = run + jnp.sum(blk, axis=0, keepdims=True)
    rank = jnp.concatenate(ranks, axis=0)
    counts = run

    padded = jnp.ceil(counts * (1.0 / BLK_M)) * BLK_M
    er = jax.lax.broadcasted_iota(jnp.int32, (NEXP, NEXP), 0)
    ec = jax.lax.broadcasted_iota(jnp.int32, (NEXP, NEXP), 1)
    ustrict = (er < ec).astype(jnp.float32)
    gs = jnp.dot(padded, ustrict, preferred_element_type=jnp.float32,
                 precision=jax.lax.Precision.HIGHEST)

    dstf = jnp.sum(oh * (gs + rank), axis=-1, keepdims=True)
    dst_ref[...] = dstf.astype(jnp.int32)
    gate_ref[...] = jnp.concatenate([g1, g2], axis=0)

    bi = (jax.lax.broadcasted_iota(jnp.int32, (NBLK, NEXP), 0)
          .astype(jnp.float32) * BLK_M)
    gsb = jnp.broadcast_to(gs, (NBLK, NEXP))
    be = jnp.sum((gsb <= bi + 0.5).astype(jnp.float32), axis=-1,
                 keepdims=True) - 1.0
    be_ref[...] = be.astype(jnp.int32)


def _router(logits):
    return pl.pallas_call(
        _router_body,
        out_shape=[
            jax.ShapeDtypeStruct((PAIRS, 1), jnp.int32),
            jax.ShapeDtypeStruct((PAIRS, 1), jnp.float32),
            jax.ShapeDtypeStruct((NBLK, 1), jnp.int32),
        ],
    )(logits)


def _make_dispatch():
    mesh = plsc.VectorSubcoreMesh(core_axis_name="c", subcore_axis_name="s")

    @functools.partial(
        pl.kernel,
        mesh=mesh,
        out_type=jax.ShapeDtypeStruct((N_SLOTS, EMSIZE), jnp.float32),
        scratch_types=[
            pltpu.VMEM((DCH, DROW), jnp.int32),
            pltpu.VMEM((2, DROW, EMSIZE), jnp.float32),
            pltpu.SemaphoreType.DMA,
        ],
    )
    def disp(h_hbm, dst_hbm, buf_hbm, idx_v, rows_v, sem):
        wid = lax.axis_index("s") * NCORE + lax.axis_index("c")
        base_pair = wid * PPT
        tok_base = base_pair % T
        pltpu.sync_copy(dst_hbm.at[pl.ds(wid * DCH, DCH)], idx_v)
        hw = [None] * DCH
        for ci in range(DCH):
            if ci >= 2:
                hw[ci - 2].wait()
            pltpu.sync_copy(h_hbm.at[pl.ds(tok_base + ci * DROW, DROW)],
                            rows_v.at[ci % 2])
            hw[ci] = pltpu.async_copy(rows_v.at[ci % 2],
                                      buf_hbm.at[idx_v.at[ci]], sem)
        hw[DCH - 2].wait()
        hw[DCH - 1].wait()

    return disp


_make_dispatch = functools.lru_cache(maxsize=1)(_make_dispatch)


def _make_combine():
    mesh = plsc.VectorSubcoreMesh(core_axis_name="c", subcore_axis_name="s")

    @functools.partial(
        pl.kernel,
        mesh=mesh,
        out_type=(
            jax.ShapeDtypeStruct((T, EMSIZE), jnp.float32),
            jax.ShapeDtypeStruct((T, EMSIZE), jnp.float32),
        ),
        scratch_types=[
            pltpu.VMEM((2 * GCH, DROW), jnp.int32),
            pltpu.VMEM((2, DROW, EMSIZE), jnp.float32),
            pltpu.SemaphoreType.DMA,
            pltpu.SemaphoreType.DMA,
        ],
    )
    def comb(y_hbm, dst_hbm, ya_hbm, yb_hbm, idx_v, rows_v, rsem, wsem):
        wid = lax.axis_index("s") * NCORE + lax.axis_index("c")
        tok_base = wid * TPT
        rowa = wid * GCH
        rowb = T // DROW + wid * GCH
        pltpu.sync_copy(dst_hbm.at[pl.ds(rowa, GCH)], idx_v.at[pl.ds(0, GCH)])
        pltpu.sync_copy(dst_hbm.at[pl.ds(rowb, GCH)],
                        idx_v.at[pl.ds(GCH, GCH)])
        nch = 2 * GCH
        hw = [None] * nch
        for ci in range(nch):
            out_hbm = ya_hbm if ci < GCH else yb_hbm
            row0 = tok_base + (ci % GCH) * DROW
            if ci >= 2:
                hw[ci - 2].wait()
            pltpu.async_copy(y_hbm.at[idx_v.at[ci]], rows_v.at[ci % 2],
                             rsem).wait()
            hw[ci] = pltpu.async_copy(rows_v.at[ci % 2],
                                      out_hbm.at[pl.ds(row0, DROW)], wsem)
        hw[nch - 2].wait()
        hw[nch - 1].wait()

    return comb


_make_combine = functools.lru_cache(maxsize=1)(_make_combine)


def _ffn_body(be_ref, x_ref, w1_ref, b1_ref, w2_ref, b2_ref, y_ref):
    hid = jnp.maximum(
        jnp.dot(x_ref[...].astype(jnp.bfloat16), w1_ref[0],
                preferred_element_type=jnp.float32)
        + b1_ref[0],
        0.0,
    )
    y_ref[...] = (
        jnp.dot(hid.astype(jnp.bfloat16), w2_ref[0],
                preferred_element_type=jnp.float32)
        + b2_ref[0]
    )


def _ffn(buf, W1, b1, W2, b2, be):
    grid_spec = pltpu.PrefetchScalarGridSpec(
        num_scalar_prefetch=1,
        grid=(NBLK,),
        in_specs=[
            pl.BlockSpec((BLK_M, EMSIZE), lambda i, be_r: (i, 0)),
            pl.BlockSpec((1, EMSIZE, NHID), lambda i, be_r: (be_r[i], 0, 0)),
            pl.BlockSpec((1, 1, NHID), lambda i, be_r: (be_r[i], 0, 0)),
            pl.BlockSpec((1, NHID, EMSIZE), lambda i, be_r: (be_r[i], 0, 0)),
            pl.BlockSpec((1, 1, EMSIZE), lambda i, be_r: (be_r[i], 0, 0)),
        ],
        out_specs=pl.BlockSpec((BLK_M, EMSIZE), lambda i, be_r: (i, 0)),
    )
    return pl.pallas_call(
        _ffn_body,
        grid_spec=grid_spec,
        out_shape=jax.ShapeDtypeStruct((N_SLOTS, EMSIZE), jnp.float32),
    )(be, buf, W1, b1.reshape(NEXP, 1, NHID), W2,
      b2.reshape(NEXP, 1, EMSIZE))


def _ln2_body(h_ref, ya_ref, yb_ref, g1_ref, g2_ref, g_ref, bb_ref, out_ref):
    z = (h_ref[...] + g1_ref[...] * ya_ref[...]
         + g2_ref[...] * yb_ref[...])
    m = jnp.mean(z, axis=-1, keepdims=True)
    cm = z - m
    v = jnp.mean(cm * cm, axis=-1, keepdims=True)
    out_ref[...] = cm * jax.lax.rsqrt(v + LN_EPS) * g_ref[...] + bb_ref[...]


def _ln2(h, ya, yb, g1, g2, g, b):
    blk = 512
    return pl.pallas_call(
        _ln2_body,
        grid=(T // blk,),
        in_specs=[
            pl.BlockSpec((blk, EMSIZE), lambda i: (i, 0)),
            pl.BlockSpec((blk, EMSIZE), lambda i: (i, 0)),
            pl.BlockSpec((blk, EMSIZE), lambda i: (i, 0)),
            pl.BlockSpec((blk, 1), lambda i: (i, 0)),
            pl.BlockSpec((blk, 1), lambda i: (i, 0)),
            pl.BlockSpec((1, EMSIZE), lambda i: (0, 0)),
            pl.BlockSpec((1, EMSIZE), lambda i: (0, 0)),
        ],
        out_specs=pl.BlockSpec((blk, EMSIZE), lambda i: (i, 0)),
        out_shape=jax.ShapeDtypeStruct((T, EMSIZE), jnp.float32),
    )(h, ya, yb, g1, g2, g.reshape(1, -1), b.reshape(1, -1))


def kernel(x, Wqkv, bqkv, Wout, bout, ln1_g, ln1_b, ln2_g, ln2_b, Wg, W1, b1,
           W2, b2):
    xt = x.reshape(T, EMSIZE)
    o = _att_fused(xt, Wqkv, bqkv)
    h, logits = _proj_ln(o, Wout, bout, xt, ln1_g, ln1_b, Wg)

    dst, gates, be = _router(logits)
    dst2 = dst.reshape(PAIRS // DROW, DROW)
    buf = _make_dispatch()(h, dst2)
    y = _ffn(buf, W1.astype(jnp.bfloat16), b1, W2.astype(jnp.bfloat16), b2,
             be.reshape(NBLK))
    ya, yb = _make_combine()(y, dst2)
    g1 = gates[:T]
    g2 = gates[T:]
    out = _ln2(h, ya, yb, g1, g2, ln2_g, ln2_b)
    return out.reshape(B, S, EMSIZE)


# speedup vs baseline: 1.0749x; 1.0749x over previous
import functools

import jax
import jax.numpy as jnp
from jax import lax
from jax.experimental import pallas as pl
from jax.experimental.pallas import tpu as pltpu
from jax.experimental.pallas import tpu_sc as plsc

EMSIZE = 1024
NHEADS = 16
NHID = 2048
NEXP = 8
B = 2
S = 2048
HD = EMSIZE // NHEADS
T = B * S
LN_EPS = 1e-5

PAIRS = 2 * T
BLK_M = 128
N_SLOTS = PAIRS + NEXP * BLK_M
NBLK = N_SLOTS // BLK_M
CH = 512

NCORE = 2
NSUB = 16
NW = NCORE * NSUB
DROW = 32
PPT = PAIRS // NW
DCH = PPT // DROW
TPT = T // NW
GCH = TPT // DROW


def _dot_t(x, w):
    return jax.lax.dot_general(
        x, w, (((1,), (1,)), ((), ())), preferred_element_type=jnp.float32)


def _qkv_body(x_ref, w_ref, b_ref, o_ref):
    o_ref[...] = _dot_t(x_ref[...], w_ref[...]) + b_ref[...]


def _qkv_proj(xt, Wqkv, bqkv):
    blk = 512
    return pl.pallas_call(
        _qkv_body,
        grid=(T // blk,),
        in_specs=[
            pl.BlockSpec((blk, EMSIZE), lambda i: (i, 0)),
            pl.BlockSpec((3 * EMSIZE, EMSIZE), lambda i: (0, 0)),
            pl.BlockSpec((1, 3 * EMSIZE), lambda i: (0, 0)),
        ],
        out_specs=pl.BlockSpec((blk, 3 * EMSIZE), lambda i: (i, 0)),
        out_shape=jax.ShapeDtypeStruct((T, 3 * EMSIZE), jnp.float32),
    )(xt, Wqkv, bqkv.reshape(1, -1))


HPB = 8


def _att_body(q_ref, k_ref, v_ref, o_ref):
    outs = []
    for j in range(HPB):
        q = q_ref[:, j * HD:(j + 1) * HD]
        k = k_ref[:, j * HD:(j + 1) * HD]
        v = v_ref[:, j * HD:(j + 1) * HD]
        att = _dot_t(q, k) * (1.0 / (HD ** 0.5))
        e = jnp.exp(att)
        s = jnp.sum(e, axis=-1, keepdims=True)
        outs.append(jnp.dot(e, v, preferred_element_type=jnp.float32) / s)
    o_ref[...] = jnp.concatenate(outs, axis=-1)


def _attention(qkv):
    blk = 512
    nblk = S // blk
    nhb = NHEADS // HPB
    return pl.pallas_call(
        _att_body,
        grid=(B, nhb, nblk),
        in_specs=[
            pl.BlockSpec((blk, HPB * HD), lambda b, h, i: (b * nblk + i, h)),
            pl.BlockSpec((S, HPB * HD), lambda b, h, i: (b, nhb + h)),
            pl.BlockSpec((S, HPB * HD), lambda b, h, i: (b, 2 * nhb + h)),
        ],
        out_specs=pl.BlockSpec((blk, HPB * HD),
                               lambda b, h, i: (b * nblk + i, h)),
        out_shape=jax.ShapeDtypeStruct((T, EMSIZE), jnp.float32),
    )(qkv, qkv, qkv)


QBLK = 512


def _att_fused_body(x_ref, wq_ref, wk_ref, wv_ref, bq_ref, bk_ref, bv_ref,
                    o_ref, k_scr, v_scr):
    i = pl.program_id(2)

    @pl.when(i == 0)
    def _kv():
        xb = x_ref[...]
        k_scr[...] = _dot_t(xb, wk_ref[...]) + bk_ref[...]
        v_scr[...] = _dot_t(xb, wv_ref[...]) + bv_ref[...]

    q = (_dot_t(x_ref[pl.ds(i * QBLK, QBLK)], wq_ref[...]) + bq_ref[...])
    outs = []
    for j in range(HPB):
        qj = q[:, j * HD:(j + 1) * HD]
        kj = k_scr[:, j * HD:(j + 1) * HD]
        vj = v_scr[:, j * HD:(j + 1) * HD]
        att = _dot_t(qj, kj) * (1.0 / (HD ** 0.5))
        e = jnp.exp(att)
        s = jnp.sum(e, axis=-1, keepdims=True)
        outs.append(jnp.dot(e, vj, preferred_element_type=jnp.float32) / s)
    o_ref[...] = jnp.concatenate(outs, axis=-1)


def _att_fused(xt, Wqkv, bqkv):
    nblk = S // QBLK
    nhb = NHEADS // HPB
    gw = HPB * HD
    b2d = bqkv.reshape(1, 3 * EMSIZE)
    return pl.pallas_call(
        _att_fused_body,
        grid=(B, nhb, nblk),
        in_specs=[
            pl.BlockSpec((S, EMSIZE), lambda b, h, i: (b, 0)),
            pl.BlockSpec((gw, EMSIZE), lambda b, h, i: (h, 0)),
            pl.BlockSpec((gw, EMSIZE), lambda b, h, i: (nhb + h, 0)),
            pl.BlockSpec((gw, EMSIZE), lambda b, h, i: (2 * nhb + h, 0)),
            pl.BlockSpec((1, gw), lambda b, h, i: (0, h)),
            pl.BlockSpec((1, gw), lambda b, h, i: (0, nhb + h)),
            pl.BlockSpec((1, gw), lambda b, h, i: (0, 2 * nhb + h)),
        ],
        out_specs=pl.BlockSpec((QBLK, gw), lambda b, h, i: (b * nblk + i, h)),
        out_shape=jax.ShapeDtypeStruct((T, EMSIZE), jnp.float32),
        scratch_shapes=[
            pltpu.VMEM((S, gw), jnp.float32),
            pltpu.VMEM((S, gw), jnp.float32),
        ],
    )(xt, Wqkv, Wqkv, Wqkv, b2d, b2d, b2d)


def _proj_ln_body(o_ref, w_ref, b_ref, x_ref, g_ref, bb_ref, wg_ref,
                  h_ref, lg_ref):
    y = _dot_t(o_ref[...], w_ref[...]) + b_ref[...] + x_ref[...]
    m = jnp.mean(y, axis=-1, keepdims=True)
    c = y - m
    v = jnp.mean(c * c, axis=-1, keepdims=True)
    h = c * jax.lax.rsqrt(v + LN_EPS) * g_ref[...] + bb_ref[...]
    h_ref[...] = h
    lg_ref[...] = jnp.dot(h, wg_ref[...], preferred_element_type=jnp.float32)


def _proj_ln(o, Wout, bout, xt, g, b, Wg):
    blk = 512
    return pl.pallas_call(
        _proj_ln_body,
        grid=(T // blk,),
        in_specs=[
            pl.BlockSpec((blk, EMSIZE), lambda i: (i, 0)),
            pl.BlockSpec((EMSIZE, EMSIZE), lambda i: (0, 0)),
            pl.BlockSpec((1, EMSIZE), lambda i: (0, 0)),
            pl.BlockSpec((blk, EMSIZE), lambda i: (i, 0)),
            pl.BlockSpec((1, EMSIZE), lambda i: (0, 0)),
            pl.BlockSpec((1, EMSIZE), lambda i: (0, 0)),
            pl.BlockSpec((EMSIZE, NEXP), lambda i: (0, 0)),
        ],
        out_specs=[
            pl.BlockSpec((blk, EMSIZE), lambda i: (i, 0)),
            pl.BlockSpec((blk, NEXP), lambda i: (i, 0)),
        ],
        out_shape=[
            jax.ShapeDtypeStruct((T, EMSIZE), jnp.float32),
            jax.ShapeDtypeStruct((T, NEXP), jnp.float32),
        ],
    )(o, Wout, bout.reshape(1, -1), xt, g.reshape(1, -1), b.reshape(1, -1),
      Wg)


def _router_body(lg_ref, dst_ref, gate_ref, be_ref):
    lg = lg_ref[...]
    mx = jnp.max(lg, axis=-1, keepdims=True)
    ex = jnp.exp(lg - mx)
    probs = ex / jnp.sum(ex, axis=-1, keepdims=True)
    cols = jax.lax.broadcasted_iota(jnp.int32, probs.shape, 1)
    v1 = jnp.max(probs, axis=-1, keepdims=True)
    i1 = jnp.min(jnp.where(probs == v1, cols, NEXP), axis=-1, keepdims=True)
    masked = jnp.where(cols == i1, -1.0, probs)
    v2 = jnp.max(masked, axis=-1, keepdims=True)
    i2 = jnp.min(jnp.where(masked == v2, cols, NEXP), axis=-1, keepdims=True)
    den = v1 + v2 + 1e-9
    g1 = v1 / den
    g2 = v2 / den

    oh1 = (cols == i1).astype(jnp.float32)
    oh2 = (cols == i2).astype(jnp.float32)
    oh = jnp.concatenate([oh1, oh2], axis=0)

    r = jax.lax.broadcasted_iota(jnp.int32, (CH, CH), 0)
    c = jax.lax.broadcasted_iota(jnp.int32, (CH, CH), 1)
    trils = (c < r).astype(jnp.float32)
    run = jnp.zeros((1, NEXP), jnp.float32)
    ranks = []
    for ci in range(PAIRS // CH):
        blk = oh[ci * CH:(ci + 1) * CH]
        ranks.append(
            jnp.dot(trils, blk, preferred_element_type=jnp.float32) + run)
        run = run + jnp.sum(blk, axis=0, keepdims=True)
    rank = jnp.concatenate(ranks, axis=0)
    counts = run

    padded = jnp.ceil(counts * (1.0 / BLK_M)) * BLK_M
    er = jax.lax.broadcasted_iota(jnp.int32, (NEXP, NEXP), 0)
    ec = jax.lax.broadcasted_iota(jnp.int32, (NEXP, NEXP), 1)
    ustrict = (er < ec).astype(jnp.float32)
    gs = jnp.dot(padded, ustrict, preferred_element_type=jnp.float32,
                 precision=jax.lax.Precision.HIGHEST)

    dstf = jnp.sum(oh * (gs + rank), axis=-1, keepdims=True)
    dst_ref[...] = dstf.astype(jnp.int32)
    gate_ref[...] = jnp.concatenate([g1, g2], axis=0)

    bi = (jax.lax.broadcasted_iota(jnp.int32, (NBLK, NEXP), 0)
          .astype(jnp.float32) * BLK_M)
    gsb = jnp.broadcast_to(gs, (NBLK, NEXP))
    be = jnp.sum((gsb <= bi + 0.5).astype(jnp.float32), axis=-1,
                 keepdims=True) - 1.0
    be_ref[...] = be.astype(jnp.int32)


def _router(logits):
    return pl.pallas_call(
        _router_body,
        out_shape=[
            jax.ShapeDtypeStruct((PAIRS, 1), jnp.int32),
            jax.ShapeDtypeStruct((PAIRS, 1), jnp.float32),
            jax.ShapeDtypeStruct((NBLK, 1), jnp.int32),
        ],
    )(logits)


def _make_dispatch():
    mesh = plsc.VectorSubcoreMesh(core_axis_name="c", subcore_axis_name="s")

    @functools.partial(
        pl.kernel,
        mesh=mesh,
        out_type=jax.ShapeDtypeStruct((N_SLOTS, EMSIZE), jnp.float32),
        scratch_types=[
            pltpu.VMEM((DCH, DROW), jnp.int32),
            pltpu.VMEM((2, DROW, EMSIZE), jnp.float32),
            pltpu.SemaphoreType.DMA,
        ],
    )
    def disp(h_hbm, dst_hbm, buf_hbm, idx_v, rows_v, sem):
        wid = lax.axis_index("s") * NCORE + lax.axis_index("c")
        base_pair = wid * PPT
        tok_base = base_pair % T
        pltpu.sync_copy(dst_hbm.at[pl.ds(wid * DCH, DCH)], idx_v)
        hw = [None] * DCH
        for ci in range(DCH):
            if ci >= 2:
                hw[ci - 2].wait()
            pltpu.sync_copy(h_hbm.at[pl.ds(tok_base + ci * DROW, DROW)],
                            rows_v.at[ci % 2])
            hw[ci] = pltpu.async_copy(rows_v.at[ci % 2],
                                      buf_hbm.at[idx_v.at[ci]], sem)
        hw[DCH - 2].wait()
        hw[DCH - 1].wait()

    return disp


_make_dispatch = functools.lru_cache(maxsize=1)(_make_dispatch)


def _make_combine():
    mesh = plsc.VectorSubcoreMesh(core_axis_name="c", subcore_axis_name="s")

    @functools.partial(
        pl.kernel,
        mesh=mesh,
        out_type=(
            jax.ShapeDtypeStruct((T, EMSIZE), jnp.float32),
            jax.ShapeDtypeStruct((T, EMSIZE), jnp.float32),
        ),
        scratch_types=[
            pltpu.VMEM((2 * GCH, DROW), jnp.int32),
            pltpu.VMEM((2, DROW, EMSIZE), jnp.float32),
            pltpu.SemaphoreType.DMA,
            pltpu.SemaphoreType.DMA,
        ],
    )
    def comb(y_hbm, dst_hbm, ya_hbm, yb_hbm, idx_v, rows_v, rsem, wsem):
        wid = lax.axis_index("s") * NCORE + lax.axis_index("c")
        tok_base = wid * TPT
        rowa = wid * GCH
        rowb = T // DROW + wid * GCH
        pltpu.sync_copy(dst_hbm.at[pl.ds(rowa, GCH)], idx_v.at[pl.ds(0, GCH)])
        pltpu.sync_copy(dst_hbm.at[pl.ds(rowb, GCH)],
                        idx_v.at[pl.ds(GCH, GCH)])
        nch = 2 * GCH
        hw = [None] * nch
        for ci in range(nch):
            out_hbm = ya_hbm if ci < GCH else yb_hbm
            row0 = tok_base + (ci % GCH) * DROW
            if ci >= 2:
                hw[ci - 2].wait()
            pltpu.async_copy(y_hbm.at[idx_v.at[ci]], rows_v.at[ci % 2],
                             rsem).wait()
            hw[ci] = pltpu.async_copy(rows_v.at[ci % 2],
                                      out_hbm.at[pl.ds(row0, DROW)], wsem)
        hw[nch - 2].wait()
        hw[nch - 1].wait()

    return comb


_make_combine = functools.lru_cache(maxsize=1)(_make_combine)


def _ffn_body(be_ref, x_ref, w1_ref, b1_ref, w2_ref, b2_ref, y_ref):
    hid = jnp.maximum(
        jnp.dot(x_ref[...].astype(jnp.bfloat16), w1_ref[0],
                preferred_element_type=jnp.float32)
        + b1_ref[0],
        0.0,
    )
    y_ref[...] = (
        jnp.dot(hid.astype(jnp.bfloat16), w2_ref[0],
                preferred_element_type=jnp.float32)
        + b2_ref[0]
    )


def _ffn(buf, W1, b1, W2, b2, be):
    grid_spec = pltpu.PrefetchScalarGridSpec(
        num_scalar_prefetch=1,
        grid=(NBLK,),
        in_specs=[
            pl.BlockSpec((BLK_M, EMSIZE), lambda i, be_r: (i, 0)),
            pl.BlockSpec((1, EMSIZE, NHID), lambda i, be_r: (be_r[i], 0, 0)),
            pl.BlockSpec((1, 1, NHID), lambda i, be_r: (be_r[i], 0, 0)),
            pl.BlockSpec((1, NHID, EMSIZE), lambda i, be_r: (be_r[i], 0, 0)),
            pl.BlockSpec((1, 1, EMSIZE), lambda i, be_r: (be_r[i], 0, 0)),
        ],
        out_specs=pl.BlockSpec((BLK_M, EMSIZE), lambda i, be_r: (i, 0)),
    )
    return pl.pallas_call(
        _ffn_body,
        grid_spec=grid_spec,
        out_shape=jax.ShapeDtypeStruct((N_SLOTS, EMSIZE), jnp.float32),
    )(be, buf, W1, b1.reshape(NEXP, 1, NHID), W2,
      b2.reshape(NEXP, 1, EMSIZE))


def _ln2_body(h_ref, ya_ref, yb_ref, g1_ref, g2_ref, g_ref, bb_ref, out_ref):
    z = (h_ref[...] + g1_ref[...] * ya_ref[...]
         + g2_ref[...] * yb_ref[...])
    m = jnp.mean(z, axis=-1, keepdims=True)
    cm = z - m
    v = jnp.mean(cm * cm, axis=-1, keepdims=True)
    out_ref[...] = cm * jax.lax.rsqrt(v + LN_EPS) * g_ref[...] + bb_ref[...]


def _ln2(h, ya, yb, g1, g2, g, b):
    blk = 512
    return pl.pallas_call(
        _ln2_body,
        grid=(T // blk,),
        in_specs=[
            pl.BlockSpec((blk, EMSIZE), lambda i: (i, 0)),
            pl.BlockSpec((blk, EMSIZE), lambda i: (i, 0)),
            pl.BlockSpec((blk, EMSIZE), lambda i: (i, 0)),
            pl.BlockSpec((blk, 1), lambda i: (i, 0)),
            pl.BlockSpec((blk, 1), lambda i: (i, 0)),
            pl.BlockSpec((1, EMSIZE), lambda i: (0, 0)),
            pl.BlockSpec((1, EMSIZE), lambda i: (0, 0)),
        ],
        out_specs=pl.BlockSpec((blk, EMSIZE), lambda i: (i, 0)),
        out_shape=jax.ShapeDtypeStruct((T, EMSIZE), jnp.float32),
    )(h, ya, yb, g1, g2, g.reshape(1, -1), b.reshape(1, -1))


def kernel(x, Wqkv, bqkv, Wout, bout, ln1_g, ln1_b, ln2_g, ln2_b, Wg, W1, b1,
           W2, b2):
    xt = x.reshape(T, EMSIZE)
    qkv = _qkv_proj(xt, Wqkv, bqkv)
    o = _attention(qkv)
    h, logits = _proj_ln(o, Wout, bout, xt, ln1_g, ln1_b, Wg)

    dst, gates, be = _router(logits)
    dst2 = dst.reshape(PAIRS // DROW, DROW)
    buf = _make_dispatch()(h, dst2)
    y = _ffn(buf, W1.astype(jnp.bfloat16), b1, W2.astype(jnp.bfloat16), b2,
             be.reshape(NBLK))
    ya, yb = _make_combine()(y, dst2)
    g1 = gates[:T]
    g2 = gates[T:]
    out = _ln2(h, ya, yb, g1, g2, ln2_g, ln2_b)
    return out.reshape(B, S, EMSIZE)
